# Initial kernel scaffold; baseline (speedup 1.0000x reference)
import jax
import jax.numpy as jnp
from jax.experimental import pallas as pl


def kernel(x, idx, val, bias):
    raise NotImplementedError("write your pallas kernel here")


# speedup vs baseline: 1.0033x; 1.0033x over previous
import functools

import jax
import jax.numpy as jnp
from jax.experimental import pallas as pl
from jax.experimental.pallas import tpu as pltpu

IN_SIZE = 2048
OUT_SIZE = 2048
BATCH = 8192

BM = 512
BN = 1024


def _mm_body(x_ref, w_ref, b_ref, o_ref):
    acc = jnp.dot(x_ref[...], w_ref[...], preferred_element_type=jnp.float32)
    o_ref[...] = jnp.maximum(acc + b_ref[...], 0.0)


@jax.jit
def _matmul_bias_relu(x, w, bias):
    grid = (BATCH // BM, OUT_SIZE // BN)
    return pl.pallas_call(
        _mm_body,
        grid=grid,
        in_specs=[
            pl.BlockSpec((BM, IN_SIZE), lambda i, j: (i, 0)),
            pl.BlockSpec((IN_SIZE, BN), lambda i, j: (0, j)),
            pl.BlockSpec((1, BN), lambda i, j: (0, j)),
        ],
        out_specs=pl.BlockSpec((BM, BN), lambda i, j: (i, j)),
        out_shape=jax.ShapeDtypeStruct((BATCH, OUT_SIZE), jnp.float32),
    )(x, w, bias.reshape(1, OUT_SIZE))


def kernel(x, idx, val, bias):
    w = jnp.zeros((IN_SIZE, OUT_SIZE), jnp.float32).at[idx[:, 0], idx[:, 1]].add(val)
    return _matmul_bias_relu(x, w, bias)


# speedup vs baseline: 1.7495x; 1.7437x over previous
import functools

import jax
import jax.numpy as jnp
from jax import lax
from jax.experimental import pallas as pl
from jax.experimental.pallas import tpu as pltpu
from jax.experimental.pallas import tpu_sc as plsc

IN_SIZE = 2048
OUT_SIZE = 2048
BATCH = 8192
NNZ = IN_SIZE * OUT_SIZE // 2

NS = 16
NC = 2
W_WORDS = IN_SIZE * OUT_SIZE
CHUNK = W_WORDS // 4
DUMP = 2048
PER_TILE = NNZ // NS
WSZ = 8192
NWIN = PER_TILE // WSZ
JROWS = WSZ // 128
SLICE = CHUNK // NS


def _sc_scatter_body(idx_ref, val_ref, out_ref, spmem, idxw, valw, locb, sem):
    c = lax.axis_index("c")
    s = lax.axis_index("s")
    ii = lax.iota(jnp.int32, 16)
    ii2 = ii * 2
    z16 = jnp.zeros((16,), jnp.float32)
    tile_base = s * PER_TILE

    for p in range(2):
        chunk_base = (c * 2 + p) * CHUNK

        def zfill(i, _):
            valw[0, pl.ds(i * 16, 16)] = z16
            return 0
        lax.fori_loop(0, WSZ // 16, zfill, 0)
        for t in range(SLICE // WSZ):
            pltpu.sync_copy(valw.at[0],
                            spmem.at[pl.ds(s * SLICE + t * WSZ, WSZ)])
        plsc.subcore_barrier()

        def window(w, _):
            slot = w % 2
            start = tile_base + w * WSZ

            @pl.when(w >= 2)
            def _():
                pltpu.make_async_copy(val_ref.at[pl.ds(0, WSZ)],
                                      valw.at[slot], sem).wait()

            pltpu.sync_copy(idx_ref.at[pl.ds(start * 2, WSZ * 2)], idxw)
            pltpu.sync_copy(val_ref.at[pl.ds(start, WSZ)], valw.at[slot])

            def group(j, _):
                for k in range(8):
                    ev = (j * 128 + k * 16) * 2 + ii2
                    rows = plsc.load_gather(idxw, [ev])
                    cols = plsc.load_gather(idxw, [ev + 1])
                    flat = rows * OUT_SIZE + cols
                    rel = flat - chunk_base
                    mine = (rel >= 0) & (rel < CHUNK)
                    loc = jnp.where(mine, rel, CHUNK + (flat & (DUMP - 1)))
                    locb[slot, j, pl.ds(k * 16, 16)] = loc
                return 0
            lax.fori_loop(0, JROWS, group, 0)

            def fire(j, _):
                pltpu.async_copy(valw.at[slot, pl.ds(j * 128, 128)],
                                 spmem.at[locb.at[slot, j]], sem, add=True)
                return 0
            lax.fori_loop(0, JROWS, fire, 0)
            return 0
        lax.fori_loop(0, NWIN, window, 0)

        pltpu.make_async_copy(val_ref.at[pl.ds(0, WSZ)], valw.at[0], sem).wait()
        pltpu.make_async_copy(val_ref.at[pl.ds(0, WSZ)], valw.at[1], sem).wait()
        plsc.subcore_barrier()

        pltpu.sync_copy(spmem.at[pl.ds(s * SLICE, SLICE)],
                        out_ref.at[pl.ds(chunk_base + s * SLICE, SLICE)])
        plsc.subcore_barrier()


@jax.jit
def _sc_scatter(idx_flat, val):
    mesh = plsc.VectorSubcoreMesh(core_axis_name="c", subcore_axis_name="s")
    return pl.kernel(
        _sc_scatter_body,
        out_type=jax.ShapeDtypeStruct((W_WORDS,), jnp.float32),
        mesh=mesh,
        scratch_types=[
            pltpu.VMEM_SHARED((CHUNK + DUMP,), jnp.float32),
            pltpu.VMEM((WSZ * 2,), jnp.int32),
            pltpu.VMEM((2, WSZ), jnp.float32),
            pltpu.VMEM((2, JROWS, 128), jnp.int32),
            pltpu.SemaphoreType.DMA,
        ],
        compiler_params=pltpu.CompilerParams(needs_layout_passes=False),
    )(idx_flat, val)


BM = 512


def _mm_body(x_ref, w_ref, b_ref, o_ref):
    xb = x_ref[...].astype(jnp.bfloat16)
    acc = jax.lax.dot_general(xb, w_ref[...], (((1,), (0,)), ((), ())),
                              preferred_element_type=jnp.float32)
    o_ref[...] = jnp.maximum(acc + b_ref[...], 0.0)


@jax.jit
def _matmul_bias_relu(x, w_bf16, bias):
    grid = (BATCH // BM,)
    return pl.pallas_call(
        _mm_body,
        grid=grid,
        in_specs=[
            pl.BlockSpec((BM, IN_SIZE), lambda i: (i, 0)),
            pl.BlockSpec((IN_SIZE, OUT_SIZE), lambda i: (0, 0)),
            pl.BlockSpec((1, OUT_SIZE), lambda i: (0, 0)),
        ],
        out_specs=pl.BlockSpec((BM, OUT_SIZE), lambda i: (i, 0)),
        out_shape=jax.ShapeDtypeStruct((BATCH, OUT_SIZE), jnp.float32),
    )(x, w_bf16, bias.reshape(1, OUT_SIZE))


def kernel(x, idx, val, bias):
    idx_flat = idx.astype(jnp.int32).reshape(-1)
    w_flat = _sc_scatter(idx_flat, val)
    w = w_flat.reshape(IN_SIZE, OUT_SIZE).astype(jnp.bfloat16)
    return _matmul_bias_relu(x, w, bias)


# speedup vs baseline: 13.1407x; 7.5113x over previous
import functools

import jax
import jax.numpy as jnp
from jax import lax
from jax.experimental import pallas as pl
from jax.experimental.pallas import tpu as pltpu
from jax.experimental.pallas import tpu_sc as plsc

IN_SIZE = 2048
OUT_SIZE = 2048
BATCH = 8192
NNZ = IN_SIZE * OUT_SIZE // 2

NS = 16
NC = 2
W_WORDS = IN_SIZE * OUT_SIZE
CHUNK = W_WORDS // 4
DUMP = 2048
PER_TILE = NNZ // NS
WSZ = 8192
NWIN = PER_TILE // WSZ
JROWS = WSZ // 128
SLICE = CHUNK // NS


def _sc_scatter_body(rows_ref, cols_ref, val_ref, out_ref,
                     spmem, rowsw, colsw, valw, locb, sem):
    c = lax.axis_index("c")
    s = lax.axis_index("s")
    z16 = jnp.zeros((16,), jnp.float32)
    tile_base = s * PER_TILE

    for p in range(2):
        chunk_base = (c * 2 + p) * CHUNK

        def zfill(i, _):
            valw[0, pl.ds(i * 16, 16)] = z16
            return 0
        lax.fori_loop(0, WSZ // 16, zfill, 0)
        for t in range(SLICE // WSZ):
            pltpu.sync_copy(valw.at[0],
                            spmem.at[pl.ds(s * SLICE + t * WSZ, WSZ)])
        plsc.subcore_barrier()

        def window(w, _):
            slot = w % 2
            start = tile_base + w * WSZ

            @pl.when(w >= 2)
            def _():
                pltpu.make_async_copy(val_ref.at[pl.ds(0, WSZ)],
                                      valw.at[slot], sem).wait()

            pltpu.sync_copy(rows_ref.at[pl.ds(start, WSZ)], rowsw)
            pltpu.sync_copy(cols_ref.at[pl.ds(start, WSZ)], colsw)
            pltpu.sync_copy(val_ref.at[pl.ds(start, WSZ)], valw.at[slot])

            def group(j, _):
                for k in range(8):
                    e = j * 128 + k * 16
                    rows = rowsw[pl.ds(e, 16)]
                    cols = colsw[pl.ds(e, 16)]
                    flat = rows * OUT_SIZE + cols
                    rel = flat - chunk_base
                    mine = (rel >= 0) & (rel < CHUNK)
                    loc = jnp.where(mine, rel, CHUNK + (flat & (DUMP - 1)))
                    locb[slot, j, pl.ds(k * 16, 16)] = loc
                return 0
            lax.fori_loop(0, JROWS, group, 0)

            def fire(j, _):
                pltpu.async_copy(valw.at[slot, pl.ds(j * 128, 128)],
                                 spmem.at[locb.at[slot, j]], sem, add=True)
                return 0
            lax.fori_loop(0, JROWS, fire, 0)
            return 0
        lax.fori_loop(0, NWIN, window, 0)

        pltpu.make_async_copy(val_ref.at[pl.ds(0, WSZ)], valw.at[0], sem).wait()
        pltpu.make_async_copy(val_ref.at[pl.ds(0, WSZ)], valw.at[1], sem).wait()
        plsc.subcore_barrier()

        pltpu.sync_copy(spmem.at[pl.ds(s * SLICE, SLICE)],
                        out_ref.at[pl.ds(chunk_base + s * SLICE, SLICE)])
        plsc.subcore_barrier()


@jax.jit
def _sc_scatter(rows, cols, val):
    mesh = plsc.VectorSubcoreMesh(core_axis_name="c", subcore_axis_name="s")
    return pl.kernel(
        _sc_scatter_body,
        out_type=jax.ShapeDtypeStruct((W_WORDS,), jnp.float32),
        mesh=mesh,
        scratch_types=[
            pltpu.VMEM_SHARED((CHUNK + DUMP,), jnp.float32),
            pltpu.VMEM((WSZ,), jnp.int32),
            pltpu.VMEM((WSZ,), jnp.int32),
            pltpu.VMEM((2, WSZ), jnp.float32),
            pltpu.VMEM((2, JROWS, 128), jnp.int32),
            pltpu.SemaphoreType.DMA,
        ],
        compiler_params=pltpu.CompilerParams(needs_layout_passes=False),
    )(rows, cols, val)


BM = 512


def _mm_body(x_ref, w_ref, b_ref, o_ref):
    xb = x_ref[...].astype(jnp.bfloat16)
    acc = jax.lax.dot_general(xb, w_ref[...], (((1,), (0,)), ((), ())),
                              preferred_element_type=jnp.float32)
    o_ref[...] = jnp.maximum(acc + b_ref[...], 0.0)


@jax.jit
def _matmul_bias_relu(x, w_bf16, bias):
    grid = (BATCH // BM,)
    return pl.pallas_call(
        _mm_body,
        grid=grid,
        in_specs=[
            pl.BlockSpec((BM, IN_SIZE), lambda i: (i, 0)),
            pl.BlockSpec((IN_SIZE, OUT_SIZE), lambda i: (0, 0)),
            pl.BlockSpec((1, OUT_SIZE), lambda i: (0, 0)),
        ],
        out_specs=pl.BlockSpec((BM, OUT_SIZE), lambda i: (i, 0)),
        out_shape=jax.ShapeDtypeStruct((BATCH, OUT_SIZE), jnp.float32),
    )(x, w_bf16, bias.reshape(1, OUT_SIZE))


def kernel(x, idx, val, bias):
    idx32 = idx.astype(jnp.int32)
    rows = idx32[:, 0]
    cols = idx32[:, 1]
    w_flat = _sc_scatter(rows, cols, val)
    w = w_flat.reshape(IN_SIZE, OUT_SIZE).astype(jnp.bfloat16)
    return _matmul_bias_relu(x, w, bias)


# speedup vs baseline: 16.0185x; 1.2190x over previous
import functools

import jax
import jax.numpy as jnp
from jax import lax
from jax.experimental import pallas as pl
from jax.experimental.pallas import tpu as pltpu
from jax.experimental.pallas import tpu_sc as plsc

IN_SIZE = 2048
OUT_SIZE = 2048
BATCH = 8192
NNZ = IN_SIZE * OUT_SIZE // 2

NS = 16
NC = 2
W_WORDS = IN_SIZE * OUT_SIZE
CHUNK = W_WORDS // 4
DUMP = 2048
PER_TILE = NNZ // NS
WSZ = 4096
NWIN = PER_TILE // WSZ
JROWS = WSZ // 128
SLICE = CHUNK // NS


def _sc_scatter_body(rows_ref, cols_ref, val_ref, out_ref,
                     spmem, rowsw, colsw, valw, locb, sem, in_sem):
    c = lax.axis_index("c")
    s = lax.axis_index("s")
    z16 = jnp.zeros((16,), jnp.float32)
    tile_base = s * PER_TILE

    def prefetch(w):
        start = tile_base + w * WSZ
        pltpu.async_copy(rows_ref.at[pl.ds(start, WSZ)], rowsw.at[w % 2],
                         in_sem)
        pltpu.async_copy(cols_ref.at[pl.ds(start, WSZ)], colsw.at[w % 2],
                         in_sem)
        pltpu.async_copy(val_ref.at[pl.ds(start, WSZ)], valw.at[w % 3],
                         in_sem)

    def drain_scatters(slot3):
        pltpu.make_async_copy(val_ref.at[pl.ds(0, WSZ)],
                              valw.at[slot3], sem).wait()

    for p in range(2):
        chunk_base = (c * 2 + p) * CHUNK

        def zfill(i, _):
            valw[0, pl.ds(i * 16, 16)] = z16
            return 0
        lax.fori_loop(0, WSZ // 16, zfill, 0)
        for t in range(SLICE // WSZ):
            pltpu.sync_copy(valw.at[0],
                            spmem.at[pl.ds(s * SLICE + t * WSZ, WSZ)])
        plsc.subcore_barrier()

        prefetch(0)

        def window(w, _):
            s2 = w % 2
            s3 = w % 3

            @pl.when(w >= 2)
            def _():
                drain_scatters((w + 1) % 3)

            @pl.when(w + 1 < NWIN)
            def _():
                prefetch(w + 1)

            start = tile_base + w * WSZ
            pltpu.make_async_copy(rows_ref.at[pl.ds(start, WSZ)],
                                  rowsw.at[s2], in_sem).wait()
            pltpu.make_async_copy(cols_ref.at[pl.ds(start, WSZ)],
                                  colsw.at[s2], in_sem).wait()
            pltpu.make_async_copy(val_ref.at[pl.ds(start, WSZ)],
                                  valw.at[s3], in_sem).wait()

            def group(j, _):
                for k in range(8):
                    e = j * 128 + k * 16
                    rows = rowsw[s2, pl.ds(e, 16)]
                    cols = colsw[s2, pl.ds(e, 16)]
                    flat = rows * OUT_SIZE + cols
                    rel = flat - chunk_base
                    mine = (rel >= 0) & (rel < CHUNK)
                    loc = jnp.where(mine, rel, CHUNK + (flat & (DUMP - 1)))
                    locb[s3, j, pl.ds(k * 16, 16)] = loc
                return 0
            lax.fori_loop(0, JROWS, group, 0)

            def fire(j, _):
                pltpu.async_copy(valw.at[s3, pl.ds(j * 128, 128)],
                                 spmem.at[locb.at[s3, j]], sem, add=True)
                return 0
            lax.fori_loop(0, JROWS, fire, 0)
            return 0
        lax.fori_loop(0, NWIN, window, 0)

        for t in range(2):
            drain_scatters(t)
        plsc.subcore_barrier()

        pltpu.sync_copy(spmem.at[pl.ds(s * SLICE, SLICE)],
                        out_ref.at[pl.ds(chunk_base + s * SLICE, SLICE)])
        plsc.subcore_barrier()


@jax.jit
def _sc_scatter(rows, cols, val):
    mesh = plsc.VectorSubcoreMesh(core_axis_name="c", subcore_axis_name="s")
    return pl.kernel(
        _sc_scatter_body,
        out_type=jax.ShapeDtypeStruct((W_WORDS,), jnp.float32),
        mesh=mesh,
        scratch_types=[
            pltpu.VMEM_SHARED((CHUNK + DUMP,), jnp.float32),
            pltpu.VMEM((2, WSZ), jnp.int32),
            pltpu.VMEM((2, WSZ), jnp.int32),
            pltpu.VMEM((3, WSZ), jnp.float32),
            pltpu.VMEM((3, JROWS, 128), jnp.int32),
            pltpu.SemaphoreType.DMA,
            pltpu.SemaphoreType.DMA,
        ],
        compiler_params=pltpu.CompilerParams(needs_layout_passes=False),
    )(rows, cols, val)


BM = 512


def _mm_body(x_ref, w_ref, b_ref, o_ref):
    xb = x_ref[...].astype(jnp.bfloat16)
    acc = jax.lax.dot_general(xb, w_ref[...], (((1,), (0,)), ((), ())),
                              preferred_element_type=jnp.float32)
    o_ref[...] = jnp.maximum(acc + b_ref[...], 0.0)


@jax.jit
def _matmul_bias_relu(x, w_bf16, bias):
    grid = (BATCH // BM,)
    return pl.pallas_call(
        _mm_body,
        grid=grid,
        in_specs=[
            pl.BlockSpec((BM, IN_SIZE), lambda i: (i, 0)),
            pl.BlockSpec((IN_SIZE, OUT_SIZE), lambda i: (0, 0)),
            pl.BlockSpec((1, OUT_SIZE), lambda i: (0, 0)),
        ],
        out_specs=pl.BlockSpec((BM, OUT_SIZE), lambda i: (i, 0)),
        out_shape=jax.ShapeDtypeStruct((BATCH, OUT_SIZE), jnp.float32),
    )(x, w_bf16, bias.reshape(1, OUT_SIZE))


def kernel(x, idx, val, bias):
    idx32 = idx.astype(jnp.int32)
    rows = idx32[:, 0]
    cols = idx32[:, 1]
    w_flat = _sc_scatter(rows, cols, val)
    w = w_flat.reshape(IN_SIZE, OUT_SIZE).astype(jnp.bfloat16)
    return _matmul_bias_relu(x, w, bias)


# speedup vs baseline: 16.5511x; 1.0332x over previous
import functools

import jax
import jax.numpy as jnp
from jax import lax
from jax.experimental import pallas as pl
from jax.experimental.pallas import tpu as pltpu
from jax.experimental.pallas import tpu_sc as plsc

IN_SIZE = 2048
OUT_SIZE = 2048
BATCH = 8192
NNZ = IN_SIZE * OUT_SIZE // 2

NS = 16
HALF_COLS = OUT_SIZE // 2
CHUNK = IN_SIZE * HALF_COLS // 2
DUMP = 2048
PER_TILE = NNZ // NS
WSZ = 4096
NWIN = PER_TILE // WSZ
JROWS = WSZ // 128
SLICE = CHUNK // NS


def _sc_scatter_body(col_base, flat_ref, val_ref, out_ref,
                     spmem, flatw, valw, locb, sem, in_sem):
    c = lax.axis_index("c")
    s = lax.axis_index("s")
    z16 = jnp.zeros((16,), jnp.float32)
    tile_base = s * PER_TILE
    row_base = c * (IN_SIZE // 2)

    def prefetch(w):
        start = tile_base + w * WSZ
        pltpu.async_copy(flat_ref.at[pl.ds(start, WSZ)], flatw.at[w % 2],
                         in_sem)
        pltpu.async_copy(val_ref.at[pl.ds(start, WSZ)], valw.at[w % 3],
                         in_sem)

    def drain_scatters(slot3):
        pltpu.make_async_copy(val_ref.at[pl.ds(0, WSZ)],
                              valw.at[slot3], sem).wait()

    def zfill(i, _):
        valw[0, pl.ds(i * 16, 16)] = z16
        return 0
    lax.fori_loop(0, WSZ // 16, zfill, 0)
    for t in range(SLICE // WSZ):
        pltpu.sync_copy(valw.at[0],
                        spmem.at[pl.ds(s * SLICE + t * WSZ, WSZ)])
    plsc.subcore_barrier()

    prefetch(0)

    def window(w, _):
        s2 = w % 2
        s3 = w % 3

        @pl.when(w >= 2)
        def _():
            drain_scatters((w + 1) % 3)

        @pl.when(w + 1 < NWIN)
        def _():
            prefetch(w + 1)

        start = tile_base + w * WSZ
        pltpu.make_async_copy(flat_ref.at[pl.ds(start, WSZ)],
                              flatw.at[s2], in_sem).wait()
        pltpu.make_async_copy(val_ref.at[pl.ds(start, WSZ)],
                              valw.at[s3], in_sem).wait()

        def group(j, _):
            for k in range(8):
                e = j * 128 + k * 16
                flat = flatw[s2, pl.ds(e, 16)]
                rl = (flat >> 11) - row_base
                cl = (flat & (OUT_SIZE - 1)) - col_base
                mine = (rl.astype(jnp.uint32) < HALF_COLS) & (
                    cl.astype(jnp.uint32) < HALF_COLS)
                loc = jnp.where(mine, rl * HALF_COLS + cl,
                                CHUNK + (flat & (DUMP - 1)))
                locb[s3, j, pl.ds(k * 16, 16)] = loc
            return 0
        lax.fori_loop(0, JROWS, group, 0)

        def fire(j, _):
            pltpu.async_copy(valw.at[s3, pl.ds(j * 128, 128)],
                             spmem.at[locb.at[s3, j]], sem, add=True)
            return 0
        lax.fori_loop(0, JROWS, fire, 0)
        return 0
    lax.fori_loop(0, NWIN, window, 0)

    for t in range(2):
        drain_scatters(t)
    plsc.subcore_barrier()

    pltpu.sync_copy(spmem.at[pl.ds(s * SLICE, SLICE)],
                    out_ref.at[pl.ds(c * CHUNK + s * SLICE, SLICE)])
    plsc.subcore_barrier()


@functools.partial(jax.jit, static_argnums=0)
def _sc_scatter(col_base, flat, val):
    mesh = plsc.VectorSubcoreMesh(core_axis_name="c", subcore_axis_name="s")
    return pl.kernel(
        functools.partial(_sc_scatter_body, col_base),
        out_type=jax.ShapeDtypeStruct((IN_SIZE * HALF_COLS,), jnp.float32),
        mesh=mesh,
        scratch_types=[
            pltpu.VMEM_SHARED((CHUNK + DUMP,), jnp.float32),
            pltpu.VMEM((2, WSZ), jnp.int32),
            pltpu.VMEM((3, WSZ), jnp.float32),
            pltpu.VMEM((3, JROWS, 128), jnp.int32),
            pltpu.SemaphoreType.DMA,
            pltpu.SemaphoreType.DMA,
        ],
        compiler_params=pltpu.CompilerParams(needs_layout_passes=False),
    )(flat, val)


BM = 512


def _mm_body_first(x_ref, w_ref, b_ref, o_ref):
    acc = jax.lax.dot_general(x_ref[...], w_ref[...],
                              (((1,), (0,)), ((), ())),
                              preferred_element_type=jnp.float32)
    o_ref[...] = jnp.maximum(acc + b_ref[...], 0.0)


def _mm_body_second(x_ref, w_ref, b_ref, z_ref, o_ref):
    del z_ref
    acc = jax.lax.dot_general(x_ref[...], w_ref[...],
                              (((1,), (0,)), ((), ())),
                              preferred_element_type=jnp.float32)
    o_ref[...] = jnp.maximum(acc + b_ref[...], 0.0)


@jax.jit
def _matmul_half_first(x_bf, w_bf, bias_half):
    return pl.pallas_call(
        _mm_body_first,
        grid=(BATCH // BM,),
        in_specs=[
            pl.BlockSpec((BM, IN_SIZE), lambda i: (i, 0)),
            pl.BlockSpec((IN_SIZE, HALF_COLS), lambda i: (0, 0)),
            pl.BlockSpec((1, HALF_COLS), lambda i: (0, 0)),
        ],
        out_specs=pl.BlockSpec((BM, HALF_COLS), lambda i: (i, 0)),
        out_shape=jax.ShapeDtypeStruct((BATCH, OUT_SIZE), jnp.float32),
    )(x_bf, w_bf, bias_half)


@jax.jit
def _matmul_half_second(x_bf, w_bf, bias_half, z_prev):
    return pl.pallas_call(
        _mm_body_second,
        grid=(BATCH // BM,),
        in_specs=[
            pl.BlockSpec((BM, IN_SIZE), lambda i: (i, 0)),
            pl.BlockSpec((IN_SIZE, HALF_COLS), lambda i: (0, 0)),
            pl.BlockSpec((1, HALF_COLS), lambda i: (0, 0)),
            pl.BlockSpec(memory_space=pl.ANY),
        ],
        out_specs=pl.BlockSpec((BM, HALF_COLS), lambda i: (i, 1)),
        out_shape=jax.ShapeDtypeStruct((BATCH, OUT_SIZE), jnp.float32),
        input_output_aliases={3: 0},
    )(x_bf, w_bf, bias_half, z_prev)


def kernel(x, idx, val, bias):
    idx32 = idx.astype(jnp.int32)
    flat = idx32[:, 0] * OUT_SIZE + idx32[:, 1]
    x_bf = x.astype(jnp.bfloat16)
    bias2 = bias.reshape(2, HALF_COLS)

    wa = _sc_scatter(0, flat, val)
    wb = _sc_scatter(HALF_COLS, flat, val)
    wa_bf = wa.reshape(IN_SIZE, HALF_COLS).astype(jnp.bfloat16)
    wb_bf = wb.reshape(IN_SIZE, HALF_COLS).astype(jnp.bfloat16)

    z = _matmul_half_first(x_bf, wa_bf, bias2[0].reshape(1, HALF_COLS))
    z = _matmul_half_second(x_bf, wb_bf, bias2[1].reshape(1, HALF_COLS), z)
    return z
